# baseline (speedup 1.0000x reference)
import functools

import jax
import jax.numpy as jnp
from jax import lax
from jax.experimental import pallas as pl
from jax.experimental.pallas import tpu as pltpu
from jax.experimental.pallas import tpu_sc as plsc

DIM = 32
BATCH = 16384
L = 16

_info = plsc.get_sparse_core_info()
NC, NS = _info.num_cores, _info.num_subcores
NW = NC * NS
BPW = BATCH // NW


def _rsqrt(x):
    i = lax.bitcast_convert_type(x, jnp.int32)
    i = jnp.int32(0x5F3759DF) - lax.shift_right_arithmetic(i, 1)
    y = lax.bitcast_convert_type(i, jnp.float32)
    for _ in range(3):
        y = y * (1.5 - 0.5 * x * y * y)
    return y


@functools.partial(
    pl.kernel,
    mesh=plsc.VectorSubcoreMesh(core_axis_name="c", subcore_axis_name="s"),
    out_type=jax.ShapeDtypeStruct((DIM, BATCH), jnp.float32),
    compiler_params=pltpu.CompilerParams(use_tc_tiling_on_sc=False),
    scratch_types=[
        pltpu.VMEM((BPW,), jnp.int32),
        pltpu.VMEM((BPW,), jnp.int32),
        pltpu.VMEM((DIM, BPW), jnp.float32),
        pltpu.VMEM((DIM, BPW), jnp.float32),
        pltpu.VMEM((DIM, BPW), jnp.float32),
        pltpu.VMEM((DIM, L), jnp.float32),
        pltpu.VMEM((DIM, L), jnp.float32),
        pltpu.SemaphoreType.DMA,
    ],
)
def _sc_embed_ln(i0_hbm, i1_hbm, t0t_hbm, t1t_hbm, gamma_hbm, beta_hbm,
                 out_hbm, idx0_v, idx1_v, r0_v, r1_v, out_v, g_v, b_v, sem):
    wid = lax.axis_index("s") * NC + lax.axis_index("c")
    base = wid * BPW

    pltpu.sync_copy(i0_hbm.at[pl.ds(base, BPW)], idx0_v)
    pltpu.sync_copy(i1_hbm.at[pl.ds(base, BPW)], idx1_v)
    pltpu.sync_copy(gamma_hbm, g_v)
    pltpu.sync_copy(beta_hbm, b_v)

    copies = []
    for d in range(DIM):
        copies.append(
            pltpu.async_copy(t0t_hbm.at[d].at[idx0_v], r0_v.at[d], sem))
        copies.append(
            pltpu.async_copy(t1t_hbm.at[d].at[idx1_v], r1_v.at[d], sem))
    for c in copies:
        c.wait()

    def body(v, carry):
        sl = pl.ds(v * L, L)
        s = jnp.zeros((L,), jnp.float32)
        q = jnp.zeros((L,), jnp.float32)
        for d in range(DIM):
            e = r0_v[d, sl] + r1_v[d, sl]
            r0_v[d, sl] = e
            s = s + e
            q = q + e * e
        mean = s * (1.0 / DIM)
        var = q * (1.0 / DIM) - mean * mean + 1e-5
        r = _rsqrt(var)
        for d in range(DIM):
            out_v[d, sl] = (r0_v[d, sl] - mean) * (r * g_v[d, :]) + b_v[d, :]
        return carry

    lax.fori_loop(0, BPW // L, body, 0)
    pltpu.sync_copy(out_v, out_hbm.at[:, pl.ds(base, BPW)])


def kernel(inputs, T0, T1, gamma, beta):
    gb = jnp.broadcast_to(gamma[:, None], (DIM, L))
    bb = jnp.broadcast_to(beta[:, None], (DIM, L))
    out_t = _sc_embed_ln(inputs[0], inputs[1], T0.T, T1.T, gb, bb)
    return out_t.T


# speedup vs baseline: 5.4965x; 5.4965x over previous
import functools

import jax
import jax.numpy as jnp
from jax import lax
from jax.experimental import pallas as pl
from jax.experimental.pallas import tpu as pltpu
from jax.experimental.pallas import tpu_sc as plsc

VOCAB = 1000000
DIM = 32
BATCH = 16384
L = 16
PACK = 128 // DIM
WAVE = 256

_info = plsc.get_sparse_core_info()
NC, NS = _info.num_cores, _info.num_subcores
NW = NC * NS
BPW = BATCH // NW


def _rsqrt(x):
    i = lax.bitcast_convert_type(x, jnp.int32)
    i = jnp.int32(0x5F3759DF) - lax.shift_right_arithmetic(i, 1)
    y = lax.bitcast_convert_type(i, jnp.float32)
    for _ in range(3):
        y = y * (1.5 - 0.5 * x * y * y)
    return y


@functools.partial(
    pl.kernel,
    mesh=plsc.VectorSubcoreMesh(core_axis_name="c", subcore_axis_name="s"),
    out_type=jax.ShapeDtypeStruct((DIM, BATCH), jnp.float32),
    compiler_params=pltpu.CompilerParams(needs_layout_passes=False),
    scratch_types=[
        pltpu.VMEM((BPW,), jnp.int32),
        pltpu.VMEM((BPW,), jnp.int32),
        pltpu.VMEM((WAVE,), jnp.int32),
        pltpu.VMEM((WAVE,), jnp.int32),
        pltpu.VMEM((WAVE, 128), jnp.float32),
        pltpu.VMEM((WAVE, 128), jnp.float32),
        pltpu.VMEM((DIM, BPW), jnp.float32),
        pltpu.VMEM((DIM, L), jnp.float32),
        pltpu.VMEM((DIM, L), jnp.float32),
        pltpu.SemaphoreType.DMA,
        pltpu.SemaphoreType.DMA,
    ],
)
def _sc_embed_ln(i0_hbm, i1_hbm, t0r_hbm, t1r_hbm, g_hbm, b_hbm,
                 out_hbm, idx0_v, idx1_v, row0_v, row1_v, r0_v, r1_v,
                 out_v, g_v, b_v, sem0, sem1):
    wid = lax.axis_index("s") * NC + lax.axis_index("c")
    base = wid * BPW

    pltpu.sync_copy(i0_hbm.at[pl.ds(base, BPW)], idx0_v)
    pltpu.sync_copy(i1_hbm.at[pl.ds(base, BPW)], idx1_v)
    pltpu.sync_copy(g_hbm, g_v)
    pltpu.sync_copy(b_hbm, b_v)

    def wave_body(w, carry):
        woff = w * WAVE

        def rows_body(g, carry2):
            sl = pl.ds(g * L, L)
            row0_v[sl] = lax.shift_right_logical(
                idx0_v[pl.ds(woff + g * L, L)], 2)
            row1_v[sl] = lax.shift_right_logical(
                idx1_v[pl.ds(woff + g * L, L)], 2)
            return carry2

        lax.fori_loop(0, WAVE // L, rows_body, 0)
        c0 = pltpu.async_copy(t0r_hbm.at[row0_v], r0_v, sem0)
        c1 = pltpu.async_copy(t1r_hbm.at[row1_v], r1_v, sem1)
        c0.wait()
        c1.wait()

        def ln_body(g, carry2):
            rows = g * L + lax.iota(jnp.int32, L)
            c0l = (idx0_v[pl.ds(woff + g * L, L)] & 3) * DIM
            c1l = (idx1_v[pl.ds(woff + g * L, L)] & 3) * DIM
            s = jnp.zeros((L,), jnp.float32)
            q = jnp.zeros((L,), jnp.float32)
            for d in range(DIM):
                e = (plsc.load_gather(r0_v, [rows, c0l + d]) +
                     plsc.load_gather(r1_v, [rows, c1l + d]))
                out_v[d, pl.ds(woff + g * L, L)] = e
                s = s + e
                q = q + e * e
            mean = s * (1.0 / DIM)
            var = q * (1.0 / DIM) - mean * mean + 1e-5
            r = _rsqrt(var)
            for d in range(DIM):
                sl = pl.ds(woff + g * L, L)
                out_v[d, sl] = (out_v[d, sl] - mean) * (r * g_v[d, :]) + b_v[d, :]
            return carry2

        lax.fori_loop(0, WAVE // L, ln_body, 0)
        return carry

    lax.fori_loop(0, BPW // WAVE, wave_body, 0)
    pltpu.sync_copy(out_v, out_hbm.at[:, pl.ds(base, BPW)])


def kernel(inputs, T0, T1, gamma, beta):
    gb = jnp.broadcast_to(gamma[:, None], (DIM, L))
    bb = jnp.broadcast_to(beta[:, None], (DIM, L))
    t0r = T0.reshape(VOCAB // PACK, 128)
    t1r = T1.reshape(VOCAB // PACK, 128)
    out_t = _sc_embed_ln(inputs[0], inputs[1], t0r, t1r, gb, bb)
    return out_t.T


# speedup vs baseline: 19.7559x; 3.5943x over previous
import functools

import jax
import jax.numpy as jnp
from jax import lax
from jax.experimental import pallas as pl
from jax.experimental.pallas import tpu as pltpu
from jax.experimental.pallas import tpu_sc as plsc

VOCAB = 1000000
DIM = 32
BATCH = 16384
L = 16
WAVE = 16

_info = plsc.get_sparse_core_info()
NC, NS = _info.num_cores, _info.num_subcores
NW = NC * NS
BPW = BATCH // NW


def _rsqrt(x):
    i = lax.bitcast_convert_type(x, jnp.int32)
    i = jnp.int32(0x5F3759DF) - lax.shift_right_arithmetic(i, 1)
    y = lax.bitcast_convert_type(i, jnp.float32)
    for _ in range(3):
        y = y * (1.5 - 0.5 * x * y * y)
    return y


@functools.partial(
    pl.kernel,
    mesh=plsc.VectorSubcoreMesh(core_axis_name="c", subcore_axis_name="s"),
    out_type=jax.ShapeDtypeStruct((DIM, BATCH), jnp.float32),
    compiler_params=pltpu.CompilerParams(needs_layout_passes=False),
    scratch_types=[
        pltpu.VMEM((BPW,), jnp.int32),
        pltpu.VMEM((BPW,), jnp.int32),
        pltpu.VMEM((WAVE * DIM, 128), jnp.float32),
        pltpu.VMEM((DIM, BPW), jnp.float32),
        pltpu.VMEM((DIM, L), jnp.float32),
        pltpu.VMEM((DIM, L), jnp.float32),
        pltpu.SemaphoreType.DMA,
    ],
)
def _sc_embed_ln(i0_hbm, i1_hbm, t0t_hbm, t1t_hbm, g_hbm, b_hbm,
                 out_hbm, idx0_v, idx1_v, blk_v,
                 out_v, g_v, b_v, sem):
    wid = lax.axis_index("s") * NC + lax.axis_index("c")
    base = wid * BPW

    pltpu.sync_copy(i0_hbm.at[pl.ds(base, BPW)], idx0_v)
    pltpu.sync_copy(i1_hbm.at[pl.ds(base, BPW)], idx1_v)
    pltpu.sync_copy(g_hbm, g_v)
    pltpu.sync_copy(b_hbm, b_v)

    def make_wave(t_hbm, idx_v, accumulate):
        def wave_body(w, carry):
            iv = idx_v[pl.ds(w * WAVE, L)]
            starts = (iv >> 7) * 128
            copies = []
            for j in range(WAVE):
                start = pl.multiple_of(jnp.squeeze(lax.slice(starts, (j,), (j + 1,))), 128)
                copies.append(pltpu.async_copy(
                    t_hbm.at[:, pl.ds(start, 128)],
                    blk_v.at[pl.ds(j * DIM, DIM), :], sem))
            for c in copies:
                c.wait()
            sl = pl.ds(w * WAVE, L)
            col = iv & 127
            rows0 = lax.iota(jnp.int32, L) * DIM
            for d in range(DIM):
                e = plsc.load_gather(blk_v, [rows0 + d, col])
                if accumulate:
                    out_v[d, sl] = out_v[d, sl] + e
                else:
                    out_v[d, sl] = e
            return carry
        return wave_body

    lax.fori_loop(0, BPW // WAVE, make_wave(t0t_hbm, idx0_v, False), 0)
    lax.fori_loop(0, BPW // WAVE, make_wave(t1t_hbm, idx1_v, True), 0)

    def ln_body(g, carry):
        sl = pl.ds(g * L, L)
        s = jnp.zeros((L,), jnp.float32)
        q = jnp.zeros((L,), jnp.float32)
        for d in range(DIM):
            e = out_v[d, sl]
            s = s + e
            q = q + e * e
        mean = s * (1.0 / DIM)
        var = q * (1.0 / DIM) - mean * mean + 1e-5
        r = _rsqrt(var)
        for d in range(DIM):
            out_v[d, sl] = (out_v[d, sl] - mean) * (r * g_v[d, :]) + b_v[d, :]
        return carry

    lax.fori_loop(0, BPW // L, ln_body, 0)
    pltpu.sync_copy(out_v, out_hbm.at[:, pl.ds(base, BPW)])


def kernel(inputs, T0, T1, gamma, beta):
    gb = jnp.broadcast_to(gamma[:, None], (DIM, L))
    bb = jnp.broadcast_to(beta[:, None], (DIM, L))
    out_t = _sc_embed_ln(inputs[0], inputs[1], T0.T, T1.T, gb, bb)
    return out_t.T


# speedup vs baseline: 19.7726x; 1.0008x over previous
import functools

import jax
import jax.numpy as jnp
from jax import lax
from jax.experimental import pallas as pl
from jax.experimental.pallas import tpu as pltpu
from jax.experimental.pallas import tpu_sc as plsc

VOCAB = 1000000
DIM = 32
BATCH = 16384
L = 16
WAVE = 8
NBUF = 2

_info = plsc.get_sparse_core_info()
NC, NS = _info.num_cores, _info.num_subcores
NW = NC * NS
BPW = BATCH // NW
NWAVES = BPW // WAVE
BUFROWS = WAVE * DIM

_DN = lax.GatherDimensionNumbers(
    offset_dims=(), collapsed_slice_dims=(0,), start_index_map=(0,))


def _perm(v, idx):
    return lax.gather(v, idx[:, None], _DN, slice_sizes=(1,),
                      mode=lax.GatherScatterMode.PROMISE_IN_BOUNDS)


def _rsqrt(x):
    i = lax.bitcast_convert_type(x, jnp.int32)
    i = jnp.int32(0x5F3759DF) - lax.shift_right_arithmetic(i, 1)
    y = lax.bitcast_convert_type(i, jnp.float32)
    for _ in range(3):
        y = y * (1.5 - 0.5 * x * y * y)
    return y


@functools.partial(
    pl.kernel,
    mesh=plsc.VectorSubcoreMesh(core_axis_name="c", subcore_axis_name="s"),
    out_type=jax.ShapeDtypeStruct((DIM, BATCH), jnp.float32),
    compiler_params=pltpu.CompilerParams(needs_layout_passes=False),
    scratch_types=[
        pltpu.VMEM((BPW,), jnp.int32),
        pltpu.VMEM((BPW,), jnp.int32),
        pltpu.VMEM((NBUF * BUFROWS, 128), jnp.float32),
        pltpu.VMEM((DIM, BPW), jnp.float32),
        pltpu.VMEM((DIM, L), jnp.float32),
        pltpu.VMEM((DIM, L), jnp.float32),
        pltpu.SemaphoreType.DMA,
        pltpu.SemaphoreType.DMA,
    ],
)
def _sc_embed_ln(i0_hbm, i1_hbm, t0t_hbm, t1t_hbm, g_hbm, b_hbm,
                 out_hbm, idx0_v, idx1_v, blk_v, out_v, g_v, b_v,
                 semA, semB):
    wid = lax.axis_index("s") * NC + lax.axis_index("c")
    base = wid * BPW
    sems = [semA, semB]

    pltpu.sync_copy(i0_hbm.at[pl.ds(base, BPW)], idx0_v)
    pltpu.sync_copy(i1_hbm.at[pl.ds(base, BPW)], idx1_v)
    pltpu.sync_copy(g_hbm, g_v)
    pltpu.sync_copy(b_hbm, b_v)

    lane = lax.iota(jnp.int32, L)
    j8 = lane & 7
    dh = lax.shift_right_logical(lane, 3)

    def fire(t_hbm, starts16, h, b):
        for j in range(WAVE):
            start = pl.multiple_of(
                jnp.squeeze(lax.slice(starts16, (h * WAVE + j,),
                                      (h * WAVE + j + 1,))), 128)
            pltpu.async_copy(
                t_hbm.at[:, pl.ds(start, 128)],
                blk_v.at[pl.ds(b * BUFROWS + j * DIM, DIM), :], sems[b])

    def drain(t_hbm, b):
        for j in range(WAVE):
            pltpu.make_async_copy(
                t_hbm.at[:, pl.ds(0, 128)],
                blk_v.at[pl.ds(b * BUFROWS + j * DIM, DIM), :],
                sems[b]).wait()

    def table_pass(t_hbm, idx_v, accumulate):
        def starts_of(g):
            return (idx_v[pl.ds(g * L, L)] >> 7) * 128

        fire(t_hbm, starts_of(0), 0, 0)

        def group_body(g, carry):
            iv16 = idx_v[pl.ds(g * L, L)]
            starts16 = (iv16 >> 7) * 128
            cols16 = iv16 & 127
            for h in range(2):
                b = h
                nb = 1 - h
                if h == 0:
                    fire(t_hbm, starts16, 1, nb)
                else:
                    @pl.when(g < NWAVES // 2 - 1)
                    def _():
                        fire(t_hbm, starts_of(g + 1), 0, nb)
                drain(t_hbm, b)
                col = _perm(cols16, h * WAVE + j8)
                ocol = g * L + h * WAVE + j8
                for d0 in range(0, DIM, 2):
                    orow = d0 + dh
                    e = plsc.load_gather(
                        blk_v, [b * BUFROWS + j8 * DIM + orow, col])
                    if accumulate:
                        e = e + plsc.load_gather(out_v, [orow, ocol])
                    plsc.store_scatter(out_v, [orow, ocol], e)
            return carry

        lax.fori_loop(0, NWAVES // 2, group_body, 0)

    table_pass(t0t_hbm, idx0_v, False)
    table_pass(t1t_hbm, idx1_v, True)

    def ln_body(g, carry):
        sl = pl.ds(g * L, L)
        s = jnp.zeros((L,), jnp.float32)
        q = jnp.zeros((L,), jnp.float32)
        for d in range(DIM):
            e = out_v[d, sl]
            s = s + e
            q = q + e * e
        mean = s * (1.0 / DIM)
        var = q * (1.0 / DIM) - mean * mean + 1e-5
        r = _rsqrt(var)
        for d in range(DIM):
            out_v[d, sl] = (out_v[d, sl] - mean) * (r * g_v[d, :]) + b_v[d, :]
        return carry

    lax.fori_loop(0, BPW // L, ln_body, 0)
    pltpu.sync_copy(out_v, out_hbm.at[:, pl.ds(base, BPW)])


def kernel(inputs, T0, T1, gamma, beta):
    gb = jnp.broadcast_to(gamma[:, None], (DIM, L))
    bb = jnp.broadcast_to(beta[:, None], (DIM, L))
    out_t = _sc_embed_ln(inputs[0], inputs[1], T0.T, T1.T, gb, bb)
    return out_t.T


# speedup vs baseline: 19.7814x; 1.0004x over previous
import functools

import jax
import jax.numpy as jnp
from jax import lax
from jax.experimental import pallas as pl
from jax.experimental.pallas import tpu as pltpu
from jax.experimental.pallas import tpu_sc as plsc

VOCAB = 1000000
DIM = 32
BATCH = 16384
L = 16
WAVE = 16

_info = plsc.get_sparse_core_info()
NC, NS = _info.num_cores, _info.num_subcores
NW = NC * NS
BPW = BATCH // NW


def _rsqrt(x):
    i = lax.bitcast_convert_type(x, jnp.int32)
    i = jnp.int32(0x5F3759DF) - lax.shift_right_arithmetic(i, 1)
    y = lax.bitcast_convert_type(i, jnp.float32)
    for _ in range(3):
        y = y * (1.5 - 0.5 * x * y * y)
    return y


@functools.partial(
    pl.kernel,
    mesh=plsc.VectorSubcoreMesh(core_axis_name="c", subcore_axis_name="s"),
    out_type=jax.ShapeDtypeStruct((DIM, BATCH), jnp.float32),
    compiler_params=pltpu.CompilerParams(needs_layout_passes=False),
    scratch_types=[
        pltpu.VMEM((BPW,), jnp.int32),
        pltpu.VMEM((BPW,), jnp.int32),
        pltpu.VMEM((WAVE * DIM, 128), jnp.float32),
        pltpu.VMEM((DIM, BPW), jnp.float32),
        pltpu.VMEM((DIM, L), jnp.float32),
        pltpu.VMEM((DIM, L), jnp.float32),
        pltpu.SemaphoreType.DMA,
    ],
)
def _sc_embed_ln(i0_hbm, i1_hbm, t0t_hbm, t1t_hbm, g_hbm, b_hbm,
                 out_hbm, idx0_v, idx1_v, blk_v,
                 out_v, g_v, b_v, sem):
    wid = lax.axis_index("s") * NC + lax.axis_index("c")
    base = wid * BPW

    pltpu.sync_copy(i0_hbm.at[pl.ds(base, BPW)], idx0_v)
    pltpu.sync_copy(i1_hbm.at[pl.ds(base, BPW)], idx1_v)
    pltpu.sync_copy(g_hbm, g_v)
    pltpu.sync_copy(b_hbm, b_v)

    def make_wave(t_hbm, idx_v, accumulate):
        def wave_body(w, carry):
            iv = idx_v[pl.ds(w * WAVE, L)]
            starts = (iv >> 7) * 128
            copies = []
            for j in range(WAVE):
                start = pl.multiple_of(jnp.squeeze(lax.slice(starts, (j,), (j + 1,))), 128)
                copies.append(pltpu.async_copy(
                    t_hbm.at[:, pl.ds(start, 128)],
                    blk_v.at[pl.ds(j * DIM, DIM), :], sem))
            for c in copies:
                c.wait()
            sl = pl.ds(w * WAVE, L)
            col = iv & 127
            rows0 = lax.iota(jnp.int32, L) * DIM
            for d in range(DIM):
                e = plsc.load_gather(blk_v, [rows0 + d, col])
                if accumulate:
                    out_v[d, sl] = out_v[d, sl] + e
                else:
                    out_v[d, sl] = e
            return carry
        return wave_body

    lax.fori_loop(0, BPW // WAVE, make_wave(t0t_hbm, idx0_v, False), 0)
    lax.fori_loop(0, BPW // WAVE, make_wave(t1t_hbm, idx1_v, True), 0)

    def ln_body(g, carry):
        sl = pl.ds(g * L, L)
        s = jnp.zeros((L,), jnp.float32)
        q = jnp.zeros((L,), jnp.float32)
        for d in range(DIM):
            e = out_v[d, sl]
            s = s + e
            q = q + e * e
        mean = s * (1.0 / DIM)
        var = q * (1.0 / DIM) - mean * mean + 1e-5
        r = _rsqrt(var)
        for d in range(DIM):
            out_v[d, sl] = (out_v[d, sl] - mean) * (r * g_v[d, :]) + b_v[d, :]
        return carry

    lax.fori_loop(0, BPW // L, ln_body, 0)
    pltpu.sync_copy(out_v, out_hbm.at[:, pl.ds(base, BPW)])


def kernel(inputs, T0, T1, gamma, beta):
    gb = jnp.broadcast_to(gamma[:, None], (DIM, L))
    bb = jnp.broadcast_to(beta[:, None], (DIM, L))
    out_t = _sc_embed_ln(inputs[0], inputs[1], T0.T, T1.T, gb, bb)
    return out_t.T
